# Initial kernel scaffold; baseline (speedup 1.0000x reference)
import jax
import jax.numpy as jnp
from jax.experimental import pallas as pl


def kernel(x, lower_index, lower_values, upper_index, upper_values, W_lower, att_lower, W_upper, att_upper, W_lin):
    raise NotImplementedError("write your pallas kernel here")


# speedup vs baseline: 12.6915x; 12.6915x over previous
import functools
import jax
import jax.numpy as jnp
from jax import lax
from jax.experimental import pallas as pl
from jax.experimental.pallas import tpu as pltpu
from jax.experimental.pallas import tpu_sc as plsc

N = 10000
E = 320000
D = 128
EPS = 1.0 + 1e-06
NP = 10240
NC = 2
NS = 16
EPT = E // NS
EPT_PAD = 20480
CH = 64
NCHUNK = EPT_PAD // CH
SLAB = 8
NSLAB = NCHUNK // SLAB
RPT = NP // NS

_f32 = jnp.float32
_i32 = jnp.int32


def _tc_pre_body(x_ref, w_ref, att_ref, xm_ref, scal_ref):
    xm = jnp.dot(x_ref[...], w_ref[...], preferred_element_type=_f32)
    xm_ref[0] = xm
    scal_ref[0] = lax.dot_general(
        att_ref[0], xm, (((0,), (1,)), ((), ())), preferred_element_type=_f32)


def _tc_pre(x_pad, w_cat, att_st):
    blk = 2048
    grid = (NP // blk, 3)
    return pl.pallas_call(
        _tc_pre_body,
        grid=grid,
        in_specs=[
            pl.BlockSpec((blk, D), lambda i, h: (i, 0)),
            pl.BlockSpec((D, D), lambda i, h: (0, h)),
            pl.BlockSpec((1, D, 2), lambda i, h: (h, 0, 0)),
        ],
        out_specs=[
            pl.BlockSpec((1, blk, D), lambda i, h: (h, i, 0)),
            pl.BlockSpec((1, 2, blk), lambda i, h: (h, 0, i)),
        ],
        out_shape=[
            jax.ShapeDtypeStruct((3, NP, D), _f32),
            jax.ShapeDtypeStruct((3, 2, NP), _f32),
        ],
    )(x_pad, w_cat, att_st)


def _sc_body(xm_st, scal_st, srcp, tgtp, valp, acc_hbm, s_hbm,
             a_src_t, a_tgt_t, s_t, src2d, tgt2d, eval2d, rows_g,
             acc_sh):
    c = lax.axis_index("c")
    s = lax.axis_index("s")

    pltpu.sync_copy(scal_st.at[c, 0], a_src_t)
    pltpu.sync_copy(scal_st.at[c, 1], a_tgt_t)

    zero16 = jnp.zeros((16,), _f32)
    iota16 = lax.iota(_i32, 16)

    def _zero_s(i, carry):
        for k in range(8):
            s_t[pl.ds(i * 128 + k * 16, 16)] = zero16
        return carry
    lax.fori_loop(0, NP // 128, _zero_s, 0)

    def _zero_rows(g, carry):
        for rr in range(8):
            for cc in range(8):
                rows_g[g * 8 + rr, pl.ds(cc * 16, 16)] = zero16
        return carry
    lax.fori_loop(0, CH // 8, _zero_rows, 0)
    for i in range(RPT // CH):
        pltpu.sync_copy(rows_g, acc_sh.at[pl.ds(s * RPT + i * CH, CH)])
    plsc.subcore_barrier()

    def _slab(j, carry):
        pltpu.sync_copy(srcp.at[c, s, pl.ds(j * SLAB, SLAB)], src2d)
        pltpu.sync_copy(tgtp.at[c, s, pl.ds(j * SLAB, SLAB)], tgt2d)
        pltpu.sync_copy(valp.at[c, s, pl.ds(j * SLAB, SLAB)], eval2d)

        def _p1(k, c1):
            for l in range(CH // 16):
                sv = src2d[k, pl.ds(l * 16, 16)]
                tv = tgt2d[k, pl.ds(l * 16, 16)]
                a = plsc.load_gather(a_src_t, [sv]) + plsc.load_gather(a_tgt_t, [tv])
                a = jnp.where(a > 0, a, a * jnp.float32(0.01))
                vv = eval2d[k, pl.ds(l * 16, 16)]
                ev = jnp.exp(a * vv)
                pos = j * (SLAB * CH) + k * CH + l * 16 + iota16
                ev = jnp.where(pos < EPT, ev, 0.0)
                eval2d[k, pl.ds(l * 16, 16)] = ev
                plsc.addupdate_scatter(s_t, [tv], ev)
            return c1
        lax.fori_loop(0, SLAB, _p1, 0)

        def _p2(k, c1):
            pltpu.sync_copy(xm_st.at[c].at[src2d.at[k]], rows_g)
            def _scale(g, c2):
                for rr in range(8):
                    r = g * 8 + rr
                    evb = plsc.load_gather(eval2d.at[k], [iota16 * 0 + r])
                    for cc in range(8):
                        rows_g[r, pl.ds(cc * 16, 16)] = (
                            rows_g[r, pl.ds(cc * 16, 16)] * evb)
                return c2
            lax.fori_loop(0, CH // 8, _scale, 0)
            pltpu.sync_copy(rows_g, acc_sh.at[tgt2d.at[k]], add=True)
            return c1
        lax.fori_loop(0, SLAB, _p2, 0)
        return carry

    lax.fori_loop(0, NSLAB, _slab, 0)

    pltpu.sync_copy(s_t, s_hbm.at[c, s])
    plsc.subcore_barrier()
    for i in range(RPT // CH):
        pltpu.sync_copy(acc_sh.at[pl.ds(s * RPT + i * CH, CH)],
                        acc_hbm.at[c, pl.ds(s * RPT + i * CH, CH)])


def _sc_edge(xm_st, scal_st, srcp, tgtp, valp):
    mesh = plsc.VectorSubcoreMesh(
        core_axis_name="c", subcore_axis_name="s", num_cores=NC, num_subcores=NS)
    return pl.kernel(
        _sc_body,
        out_type=[
            jax.ShapeDtypeStruct((NC, NP, D), _f32),
            jax.ShapeDtypeStruct((NC, NS, NP), _f32),
        ],
        mesh=mesh,
        compiler_params=pltpu.CompilerParams(needs_layout_passes=False),
        scratch_types=[
            pltpu.VMEM((NP,), _f32),
            pltpu.VMEM((NP,), _f32),
            pltpu.VMEM((NP,), _f32),
            pltpu.VMEM((SLAB, CH), _i32),
            pltpu.VMEM((SLAB, CH), _i32),
            pltpu.VMEM((SLAB, CH), _f32),
            pltpu.VMEM((CH, D), _f32),
            pltpu.VMEM_SHARED((NP, D), _f32),
        ],
    )(xm_st, scal_st, srcp, tgtp, valp)


def _combine_body(acc_ref, s_ref, skip_ref, out_ref):
    eps16 = jnp.float32(1e-16)
    sl = jnp.sum(s_ref[0], axis=0)[:, None] + eps16
    su = jnp.sum(s_ref[1], axis=0)[:, None] + eps16
    out_ref[...] = jnp.maximum(
        acc_ref[0] / sl + acc_ref[1] / su + skip_ref[0], 0.0)


def _combine(acc, s_part, xm_st):
    blk = 2048
    return pl.pallas_call(
        _combine_body,
        grid=(NP // blk,),
        in_specs=[
            pl.BlockSpec((2, blk, D), lambda i: (0, i, 0)),
            pl.BlockSpec((2, NS, blk), lambda i: (0, 0, i)),
            pl.BlockSpec((1, blk, D), lambda i: (2, i, 0)),
        ],
        out_specs=pl.BlockSpec((blk, D), lambda i: (i, 0)),
        out_shape=jax.ShapeDtypeStruct((NP, D), _f32),
    )(acc, s_part, xm_st)


def kernel(x, lower_index, lower_values, upper_index, upper_values,
           W_lower, att_lower, W_upper, att_upper, W_lin):
    x_pad = jnp.pad(x, ((0, NP - N), (0, 0)))
    w_cat = jnp.concatenate([W_lower, W_upper, EPS * W_lin], axis=1)
    att_st = jnp.stack([
        att_lower.reshape(2, D).T,
        att_upper.reshape(2, D).T,
        jnp.zeros((D, 2), _f32),
    ])
    xm_st, scal_st = _tc_pre(x_pad, w_cat, att_st)

    def _prep(arr):
        a = arr.reshape(NS, EPT)
        a = jnp.pad(a, ((0, 0), (0, EPT_PAD - EPT)))
        return a.reshape(NS, NCHUNK, CH)

    srcp = jnp.stack([_prep(lower_index[1]), _prep(upper_index[1])])
    tgtp = jnp.stack([_prep(lower_index[0]), _prep(upper_index[0])])
    valp = jnp.stack([_prep(lower_values.astype(_f32)),
                      _prep(upper_values.astype(_f32))])

    acc, s_part = _sc_edge(xm_st, scal_st, srcp, tgtp, valp)
    out = _combine(acc, s_part, xm_st)
    return out[:N]
